# baseline (speedup 1.0000x reference)
import math

import jax
import jax.numpy as jnp
from jax.experimental import pallas as pl
from jax.experimental.pallas import tpu as pltpu

_S = 2048
_HD = 128
_DL = 16
_TOPK = 8
_TILE = 512
_NT = _S // _TILE
_SC2 = 1.0 / math.sqrt(_DL)
_NEG_INF = float("-inf")


def _dot_t(a, b):
    return jax.lax.dot_general(
        a, b, (((1,), (1,)), ((), ())), preferred_element_type=jnp.float32
    )


def _attn_kernel(q_ref, k_ref, v_ref, vl_ref,
                 wql_ref, bql_ref, wkl_ref, bkl_ref,
                 wqh_ref, bqh_ref, wkh_ref, bkh_ref,
                 out_ref, s_ref, sh_ref):
    q = q_ref[0]
    k = k_ref[0]
    vl = vl_ref[0]

    q_low = _dot_t(q, wql_ref[...]) + bql_ref[...]
    k_low = _dot_t(k, wkl_ref[...]) + bkl_ref[...]
    q_high = _dot_t(q, wqh_ref[...]) + bqh_ref[...]
    k_high = _dot_t(k, wkh_ref[...]) + bkh_ref[...]

    rows = jax.lax.broadcasted_iota(jnp.int32, (_S, _TILE), 0)
    thr = jnp.full((_S, 1), _NEG_INF, dtype=jnp.float32)
    for t in range(_NT):
        lo = t * _TILE
        s_tile = _dot_t(q_low, k_low[lo:lo + _TILE, :]) * _SC2
        cond = rows == vl[:, lo:lo + _TILE]
        s_tile = jnp.where(cond, _NEG_INF, s_tile)
        s_ref[:, lo:lo + _TILE] = s_tile
        thr = jnp.maximum(thr, jnp.max(s_tile, axis=1, keepdims=True))

    for p in range(_TOPK - 1):
        if p < _NT:
            lo = p * _TILE
            sh_ref[:, lo:lo + _TILE] = _dot_t(
                q_high, k_high[lo:lo + _TILE, :]) * _SC2
        m = jnp.full((_S, 1), _NEG_INF, dtype=jnp.float32)
        for t in range(_NT):
            lo = t * _TILE
            tile = s_ref[:, lo:lo + _TILE]
            cand = jnp.where(tile < thr, tile, _NEG_INF)
            m = jnp.maximum(m, jnp.max(cand, axis=1, keepdims=True))
        thr = m

    acc = jnp.zeros((_S, _HD), dtype=jnp.float32)
    for t in range(_NT):
        lo = t * _TILE
        tile = s_ref[:, lo:lo + _TILE]
        corr = jnp.where(tile >= thr, sh_ref[:, lo:lo + _TILE], tile)
        cmax = jnp.max(corr, axis=0, keepdims=True)
        e = jnp.exp(corr - cmax)
        csum = jnp.sum(e, axis=0, keepdims=True)
        attn = e / csum
        acc = acc + jnp.dot(attn, v_ref[0][lo:lo + _TILE, :],
                            preferred_element_type=jnp.float32)
    out_ref[0] = acc


def kernel(queries, keys, values, valid_lens, Wq_low, bq_low, Wk_low, bk_low,
           Wq_high, bq_high, Wk_high, bk_high):
    b, s, hd = queries.shape
    vl3 = jnp.clip(valid_lens, 0, s - 1).reshape(b, 1, s)
    bql = bq_low.reshape(1, _DL)
    bkl = bk_low.reshape(1, _DL)
    bqh = bq_high.reshape(1, _DL)
    bkh = bk_high.reshape(1, _DL)

    full = lambda shape: pl.BlockSpec(shape, lambda i: (0,) * len(shape))
    per_b = lambda shape: pl.BlockSpec(shape, lambda i: (i,) + (0,) * (len(shape) - 1))

    return pl.pallas_call(
        _attn_kernel,
        grid=(b,),
        in_specs=[
            per_b((1, _S, _HD)),
            per_b((1, _S, _HD)),
            per_b((1, _S, _HD)),
            per_b((1, 1, _S)),
            full((_DL, _HD)), full((1, _DL)),
            full((_DL, _HD)), full((1, _DL)),
            full((_DL, _HD)), full((1, _DL)),
            full((_DL, _HD)), full((1, _DL)),
        ],
        out_specs=per_b((1, _S, _HD)),
        out_shape=jax.ShapeDtypeStruct((b, _S, _HD), jnp.float32),
        scratch_shapes=[pltpu.VMEM((_S, _S), jnp.float32),
                        pltpu.VMEM((_S, _S), jnp.float32)],
    )(queries, keys, values, vl3, Wq_low, bql, Wk_low, bkl,
      Wq_high, bqh, Wk_high, bkh)


# speedup vs baseline: 1.0315x; 1.0315x over previous
import math

import jax
import jax.numpy as jnp
from jax.experimental import pallas as pl
from jax.experimental.pallas import tpu as pltpu

_S = 2048
_HD = 128
_DL = 16
_TOPK = 8
_TILE = 512
_NT = _S // _TILE
_SC2 = 1.0 / math.sqrt(_DL)
_NEG_INF = float("-inf")


def _dot_t(a, b):
    return jax.lax.dot_general(
        a, b, (((1,), (1,)), ((), ())), preferred_element_type=jnp.float32
    )


def _attn_kernel(q_ref, k_ref, v_ref, vl_ref,
                 wql_ref, bql_ref, wkl_ref, bkl_ref,
                 wqh_ref, bqh_ref, wkh_ref, bkh_ref,
                 out_ref, s_ref):
    q = q_ref[0]
    k = k_ref[0]
    vl = vl_ref[0]

    q_low = _dot_t(q, wql_ref[...]) + bql_ref[...]
    k_low = _dot_t(k, wkl_ref[...]) + bkl_ref[...]
    q_high = (_dot_t(q, wqh_ref[...]) + bqh_ref[...]).astype(jnp.bfloat16)
    k_high = (_dot_t(k, wkh_ref[...]) + bkh_ref[...]).astype(jnp.bfloat16)

    rows = jax.lax.broadcasted_iota(jnp.int32, (_S, _TILE), 0)
    thr = jnp.full((_S, 1), _NEG_INF, dtype=jnp.float32)
    for t in range(_NT):
        lo = t * _TILE
        s_tile = _dot_t(q_low, k_low[lo:lo + _TILE, :]) * _SC2
        cond = rows == vl[:, lo:lo + _TILE]
        s_tile = jnp.where(cond, _NEG_INF, s_tile)
        s_ref[:, lo:lo + _TILE] = s_tile
        thr = jnp.maximum(thr, jnp.max(s_tile, axis=1, keepdims=True))

    for p in range(_TOPK - 1):
        m = jnp.full((_S, 1), _NEG_INF, dtype=jnp.float32)
        for t in range(_NT):
            lo = t * _TILE
            tile = s_ref[:, lo:lo + _TILE]
            cand = jnp.where(tile < thr, tile, _NEG_INF)
            m = jnp.maximum(m, jnp.max(cand, axis=1, keepdims=True))
        thr = m

    acc = jnp.zeros((_S, _HD), dtype=jnp.float32)
    for t in range(_NT):
        lo = t * _TILE
        tile = s_ref[:, lo:lo + _TILE]
        sh = _dot_t(q_high, k_high[lo:lo + _TILE, :]).astype(jnp.float32) * _SC2
        corr = jnp.where(tile >= thr, sh, tile)
        cmax = jnp.max(corr, axis=0, keepdims=True)
        e = jnp.exp(corr - cmax)
        csum = jnp.sum(e, axis=0, keepdims=True)
        attn = (e / csum).astype(jnp.bfloat16)
        acc = acc + jnp.dot(attn, v_ref[0][lo:lo + _TILE, :].astype(jnp.bfloat16),
                            preferred_element_type=jnp.float32)
    out_ref[0] = acc


def kernel(queries, keys, values, valid_lens, Wq_low, bq_low, Wk_low, bk_low,
           Wq_high, bq_high, Wk_high, bk_high):
    b, s, hd = queries.shape
    vl3 = jnp.clip(valid_lens, 0, s - 1).reshape(b, 1, s)
    bql = bq_low.reshape(1, _DL)
    bkl = bk_low.reshape(1, _DL)
    bqh = bq_high.reshape(1, _DL)
    bkh = bk_high.reshape(1, _DL)

    full = lambda shape: pl.BlockSpec(shape, lambda i: (0,) * len(shape))
    per_b = lambda shape: pl.BlockSpec(shape, lambda i: (i,) + (0,) * (len(shape) - 1))

    return pl.pallas_call(
        _attn_kernel,
        grid=(b,),
        in_specs=[
            per_b((1, _S, _HD)),
            per_b((1, _S, _HD)),
            per_b((1, _S, _HD)),
            per_b((1, 1, _S)),
            full((_DL, _HD)), full((1, _DL)),
            full((_DL, _HD)), full((1, _DL)),
            full((_DL, _HD)), full((1, _DL)),
            full((_DL, _HD)), full((1, _DL)),
        ],
        out_specs=per_b((1, _S, _HD)),
        out_shape=jax.ShapeDtypeStruct((b, _S, _HD), jnp.float32),
        scratch_shapes=[pltpu.VMEM((_S, _S), jnp.float32)],
    )(queries, keys, values, vl3, Wq_low, bql, Wk_low, bkl,
      Wq_high, bqh, Wk_high, bkh)


# speedup vs baseline: 1.0665x; 1.0339x over previous
import math

import jax
import jax.numpy as jnp
from jax.experimental import pallas as pl
from jax.experimental.pallas import tpu as pltpu

_S = 2048
_HD = 128
_DL = 16
_TOPK = 8
_TILE = 512
_NT = _S // _TILE
_SC2 = 1.0 / math.sqrt(_DL)
_NEG_INF = float("-inf")


def _dot_t(a, b):
    return jax.lax.dot_general(
        a, b, (((1,), (1,)), ((), ())), preferred_element_type=jnp.float32
    )


def _attn_kernel(q_ref, k_ref, v_ref, vl_ref,
                 wql_ref, bql_ref, wkl_ref, bkl_ref,
                 wqh_ref, bqh_ref, wkh_ref, bkh_ref,
                 out_ref, s_ref):
    q = q_ref[0]
    k = k_ref[0]
    vl = vl_ref[0]

    q_low = (_dot_t(q, wql_ref[...]) + bql_ref[...]) * _SC2
    k_low = _dot_t(k, wkl_ref[...]) + bkl_ref[...]
    q_high = (_dot_t(q, wqh_ref[...]) + bqh_ref[...]) * _SC2
    k_high = _dot_t(k, wkh_ref[...]) + bkh_ref[...]

    rows = jax.lax.broadcasted_iota(jnp.int32, (_S, _TILE), 0)
    thr = jnp.full((_S, 1), _NEG_INF, dtype=jnp.float32)
    for t in range(_NT):
        lo = t * _TILE
        s_tile = _dot_t(q_low, k_low[lo:lo + _TILE, :])
        cond = rows == vl[:, lo:lo + _TILE]
        s_tile = jnp.where(cond, _NEG_INF, s_tile)
        s_ref[:, lo:lo + _TILE] = s_tile
        thr = jnp.maximum(thr, jnp.max(s_tile, axis=1, keepdims=True))

    for p in range(_TOPK - 1):
        m = jnp.full((_S, 1), _NEG_INF, dtype=jnp.float32)
        for t in range(_NT):
            lo = t * _TILE
            tile = s_ref[:, lo:lo + _TILE]
            cand = jnp.where(tile < thr, tile, _NEG_INF)
            m = jnp.maximum(m, jnp.max(cand, axis=1, keepdims=True))
        thr = m

    acc = jnp.zeros((_S, _HD), dtype=jnp.float32)
    for t in range(_NT):
        lo = t * _TILE
        tile = s_ref[:, lo:lo + _TILE]
        sh = _dot_t(q_high, k_high[lo:lo + _TILE, :])
        corr = jnp.where(tile >= thr, sh, tile)
        cmax = jnp.max(corr, axis=0, keepdims=True)
        e = jnp.exp(corr - cmax)
        csum = jnp.sum(e, axis=0, keepdims=True)
        attn = e / csum
        acc = acc + jnp.dot(attn, v_ref[0][lo:lo + _TILE, :],
                            preferred_element_type=jnp.float32)
    out_ref[0] = acc


def kernel(queries, keys, values, valid_lens, Wq_low, bq_low, Wk_low, bk_low,
           Wq_high, bq_high, Wk_high, bk_high):
    b, s, hd = queries.shape
    vl3 = jnp.clip(valid_lens, 0, s - 1).reshape(b, 1, s)
    bql = bq_low.reshape(1, _DL)
    bkl = bk_low.reshape(1, _DL)
    bqh = bq_high.reshape(1, _DL)
    bkh = bk_high.reshape(1, _DL)

    full = lambda shape: pl.BlockSpec(shape, lambda i: (0,) * len(shape))
    per_b = lambda shape: pl.BlockSpec(shape, lambda i: (i,) + (0,) * (len(shape) - 1))

    return pl.pallas_call(
        _attn_kernel,
        grid=(b,),
        in_specs=[
            per_b((1, _S, _HD)),
            per_b((1, _S, _HD)),
            per_b((1, _S, _HD)),
            per_b((1, 1, _S)),
            full((_DL, _HD)), full((1, _DL)),
            full((_DL, _HD)), full((1, _DL)),
            full((_DL, _HD)), full((1, _DL)),
            full((_DL, _HD)), full((1, _DL)),
        ],
        out_specs=per_b((1, _S, _HD)),
        out_shape=jax.ShapeDtypeStruct((b, _S, _HD), jnp.float32),
        scratch_shapes=[pltpu.VMEM((_S, _S), jnp.float32)],
    )(queries, keys, values, vl3, Wq_low, bql, Wk_low, bkl,
      Wq_high, bqh, Wk_high, bkh)
